# speedup vs baseline: 2.8538x; 2.8538x over previous
import jax
import jax.numpy as jnp
from jax.experimental import pallas as pl
from jax.experimental.pallas import tpu as pltpu

N_CLASSES = 100000
ROWS = 256
CHUNK = 12544
NCHUNK = 8


def _lse_gather_kernel(x_ref, logits_ref, out_ref, m_ref, s_ref, g_ref):
    c = pl.program_id(0)

    @pl.when(c == 0)
    def _init():
        m_ref[...] = jnp.full((ROWS, 1), -jnp.inf, jnp.float32)
        s_ref[...] = jnp.zeros((ROWS, 1), jnp.float32)
        g_ref[...] = jnp.zeros((ROWS, 1), jnp.float32)

    v = logits_ref[...]
    col = c * CHUNK + jax.lax.broadcasted_iota(jnp.int32, (ROWS, CHUNK), 1)
    vm = jnp.where(col < N_CLASSES, v, -jnp.inf)

    m_old = m_ref[...]
    cmax = jnp.max(vm, axis=1, keepdims=True)
    m_new = jnp.maximum(m_old, cmax)
    s_ref[...] = s_ref[...] * jnp.exp(m_old - m_new) + jnp.sum(
        jnp.exp(vm - m_new), axis=1, keepdims=True
    )
    m_ref[...] = m_new

    g_ref[...] += jnp.sum(
        jnp.where(col == x_ref[...], v, 0.0), axis=1, keepdims=True
    )

    @pl.when(c == NCHUNK - 1)
    def _fin():
        out_ref[...] = g_ref[...] - m_ref[...] - jnp.log(s_ref[...])


def _run(x2, logits2, interpret=False):
    return pl.pallas_call(
        _lse_gather_kernel,
        grid=(NCHUNK,),
        in_specs=[
            pl.BlockSpec((ROWS, 1), lambda c: (0, 0)),
            pl.BlockSpec((ROWS, CHUNK), lambda c: (0, c)),
        ],
        out_specs=pl.BlockSpec((ROWS, 1), lambda c: (0, 0)),
        out_shape=jax.ShapeDtypeStruct((ROWS, 1), jnp.float32),
        scratch_shapes=[
            pltpu.VMEM((ROWS, 1), jnp.float32),
            pltpu.VMEM((ROWS, 1), jnp.float32),
            pltpu.VMEM((ROWS, 1), jnp.float32),
        ],
        interpret=interpret,
    )(x2, logits2)


def kernel(x, logits):
    logits2 = logits.reshape(ROWS, N_CLASSES)
    x2 = x.reshape(ROWS, 1).astype(jnp.int32)
    out = _run(x2, logits2)
    return out.reshape(x.shape)
